# baseline (speedup 1.0000x reference)
import functools

import jax
import jax.numpy as jnp
from jax import lax
from jax.experimental import pallas as pl
from jax.experimental.pallas import tpu as pltpu
from jax.experimental.pallas import tpu_sc as plsc

THRES_STATIC = 0.001
THRES_DIST = 0.002

B, N, M = 4, 4096, 4096
MT = 512
L = 16
NW = 32
MSC = 512
MW = MSC * B // NW
SUB = MW // L
NSUB = 1
CH = 512


def _rne_bf16(v):
    t = v * 65537.0
    return t - (t - v)


def _sc_dist2_body(xs_hbm, yc_hbm, out_hbm, yv, xb, wb, av, dv):
    wid = lax.axis_index("s") * 2 + lax.axis_index("c")
    b = wid // (NW // B)
    part = wid % (NW // B)
    m0 = part * MW

    for c in range(3):
        pltpu.sync_copy(yc_hbm.at[pl.ds((b * 3 + c) * M + m0, MW)],
                        yv.at[pl.ds(c * MW, MW)])

    def yprep(j, _):
        o = j * L
        y0 = yv[pl.ds(o, L)]
        y1 = yv[pl.ds(MW + o, L)]
        y2 = yv[pl.ds(2 * MW + o, L)]
        dv[pl.ds(o, L)] = y0 * y0 + y1 * y1 + y2 * y2
        yv[pl.ds(o, L)] = _rne_bf16(y0)
        yv[pl.ds(MW + o, L)] = _rne_bf16(y1)
        yv[pl.ds(2 * MW + o, L)] = _rne_bf16(y2)
        return 0

    lax.fori_loop(0, MW // L, yprep, 0)

    inf16 = jnp.full((L,), jnp.inf, jnp.float32)

    def ainit(i, _):
        av[pl.ds(i * L, L)] = inf16
        return 0

    lax.fori_loop(0, NSUB * SUB, ainit, 0)

    for cc in range(N // CH):
        pltpu.sync_copy(
            xs_hbm.at[pl.ds((b * N + cc * CH) * 3 * L, CH * 3 * L)], xb)

        def xprep(j, _):
            o = j * 3 * L
            q = j * 4 * L
            a0 = xb[pl.ds(o, L)]
            a1 = xb[pl.ds(o + L, L)]
            a2 = xb[pl.ds(o + 2 * L, L)]
            wb[pl.ds(q + 3 * L, L)] = a0 * a0 + a1 * a1 + a2 * a2
            wb[pl.ds(q, L)] = -2.0 * _rne_bf16(a0)
            wb[pl.ds(q + L, L)] = -2.0 * _rne_bf16(a1)
            wb[pl.ds(q + 2 * L, L)] = -2.0 * _rne_bf16(a2)
            return 0

        lax.fori_loop(0, CH, xprep, 0)

        for sg in range(NSUB):
            sb = sg * SUB * L
            ys = []
            for k in range(SUB):
                ys.append((yv[pl.ds(sb + k * L, L)],
                           yv[pl.ds(MW + sb + k * L, L)],
                           yv[pl.ds(2 * MW + sb + k * L, L)]))

            def inner(j, accs):
                q = j * 4 * L
                w0 = wb[pl.ds(q, L)]
                w1 = wb[pl.ds(q + L, L)]
                w2 = wb[pl.ds(q + 2 * L, L)]
                u = wb[pl.ds(q + 3 * L, L)]
                out = []
                for k in range(SUB):
                    y0, y1, y2 = ys[k]
                    t = u + w0 * y0 + w1 * y1 + w2 * y2
                    out.append(jnp.minimum(accs[k], t))
                return tuple(out)

            accs = lax.fori_loop(
                0, CH, inner,
                tuple(av[pl.ds(sb + k * L, L)] for k in range(SUB)))
            for k in range(SUB):
                av[pl.ds(sb + k * L, L)] = accs[k]

    def fin(i, _):
        o = i * L
        dv[pl.ds(o, L)] = jnp.maximum(av[pl.ds(o, L)] + dv[pl.ds(o, L)], 0.0)
        return 0

    lax.fori_loop(0, MW // L, fin, 0)
    pltpu.sync_copy(dv, out_hbm.at[pl.ds(b * MSC + m0, MW)])


def _sc_dist2(xs, yc):
    mesh = plsc.VectorSubcoreMesh(core_axis_name="c", subcore_axis_name="s")
    f = pl.kernel(
        _sc_dist2_body,
        out_type=jax.ShapeDtypeStruct((B * MSC,), jnp.float32),
        mesh=mesh,
        scratch_types=[
            pltpu.VMEM((3 * MW,), jnp.float32),
            pltpu.VMEM((CH * 3 * L,), jnp.float32),
            pltpu.VMEM((CH * 4 * L,), jnp.float32),
            pltpu.VMEM((NSUB * SUB * L,), jnp.float32),
            pltpu.VMEM((MW,), jnp.float32),
        ],
    )
    return f(xs, yc)


def _dist2_body(x_ref, w_ref, y_ref, y8_ref, out_ref):
    a = x_ref[0]
    yb = y_ref[0]
    aa = jnp.sum(a * a, axis=1, keepdims=True)
    ab = jax.lax.dot_general(
        w_ref[0], y8_ref[0],
        (((1,), (0,)), ((), ())),
        preferred_element_type=jnp.float32,
    )
    t = ab + aa
    tmin = jnp.min(t, axis=0, keepdims=True)
    bb = jnp.sum(yb * yb, axis=0, keepdims=True)
    out_ref[...] = jnp.maximum(tmin + bb, 0.0)[None]


def _loss_body(d2_ref, y0_ref, y1_ref, sdyn_ref, cdyn_ref, sst_ref, cst_ref):
    d2 = d2_ref[...]
    pos = d2 < THRES_DIST
    neg = d2 > THRES_STATIC
    sq = jnp.zeros_like(d2)
    sst = jnp.float32(0.0)
    cst = jnp.float32(0.0)
    for c in range(3):
        diff = y1_ref[c * B:(c + 1) * B, :] - y0_ref[c * B:(c + 1) * B, :]
        sq = sq + diff * diff
        sel_st = jnp.logical_and(neg, diff > 0.0)
        sst = sst + jnp.sum(jnp.where(sel_st, diff * diff, 0.0))
        cst = cst + jnp.sum(sel_st.astype(jnp.float32))
    sel_dyn = jnp.logical_and(pos, sq > 0.0)
    norm = jnp.sqrt(jnp.where(sel_dyn, sq, 1.0))
    z = 0.1 - norm
    sp = jnp.maximum(z, 0.0) + jnp.log1p(jnp.exp(-jnp.abs(z)))
    sdyn_ref[0, 0] = jnp.sum(jnp.where(sel_dyn, sp, 0.0))
    cdyn_ref[0, 0] = jnp.sum(sel_dyn.astype(jnp.float32))
    sst_ref[0, 0] = sst
    cst_ref[0, 0] = cst


@jax.jit
def kernel(x, y_hat0, y_hat1):
    xp = x[:, 0, :, :3].astype(jnp.float32)
    y0p = y_hat0[:, 0].astype(jnp.float32)
    yt = jnp.transpose(y0p, (0, 2, 1))
    wp = jnp.zeros((B, N, 8), jnp.bfloat16).at[:, :, :3].set(
        (-2.0 * xp).astype(jnp.bfloat16))
    y8 = jnp.zeros((B, 8, M), jnp.bfloat16).at[:, :3, :].set(
        yt.astype(jnp.bfloat16))

    xs = jnp.broadcast_to(xp[..., None], (B, N, 3, L)).reshape(-1)
    ycm = jnp.transpose(y0p, (0, 2, 1)).reshape(-1)
    d2sc = _sc_dist2(xs, ycm).reshape(B, MSC)

    MR = M - MSC
    d2tc = pl.pallas_call(
        _dist2_body,
        grid=(B, MR // MT),
        in_specs=[
            pl.BlockSpec((1, N, 3), lambda b, m: (b, 0, 0)),
            pl.BlockSpec((1, N, 8), lambda b, m: (b, 0, 0)),
            pl.BlockSpec((1, 3, MT), lambda b, m: (b, 0, m + MSC // MT)),
            pl.BlockSpec((1, 8, MT), lambda b, m: (b, 0, m + MSC // MT)),
        ],
        out_specs=pl.BlockSpec((1, 1, MT), lambda b, m: (b, 0, m)),
        out_shape=jax.ShapeDtypeStruct((B, 1, MR), jnp.float32),
    )(xp, wp, yt, y8)
    d2 = jnp.concatenate([d2sc, d2tc[:, 0, :]], axis=1)

    y0r = jnp.transpose(y0p, (2, 0, 1)).reshape(3 * B, M)
    y1r = jnp.transpose(y_hat1[:, 0].astype(jnp.float32), (2, 0, 1)).reshape(3 * B, M)

    scal = jax.ShapeDtypeStruct((1, 1), jnp.float32)
    sspec = pl.BlockSpec(memory_space=pltpu.SMEM)
    sdyn, cdyn, sst, cst = pl.pallas_call(
        _loss_body,
        out_shape=(scal, scal, scal, scal),
        out_specs=(sspec, sspec, sspec, sspec),
    )(d2, y0r, y1r)

    loss_dynamic = (sdyn[0, 0] / cdyn[0, 0]).astype(jnp.float32)
    loss_static = (sst[0, 0] / cst[0, 0]).astype(jnp.float32)
    return (loss_dynamic, loss_static)


# speedup vs baseline: 2.5158x; 2.5158x over previous
import functools

import jax
import jax.numpy as jnp
from jax.experimental import pallas as pl
from jax.experimental.pallas import tpu as pltpu

THRES_STATIC = 0.001
THRES_DIST = 0.002

B, N, M = 4, 4096, 4096
MT = 1024


def _dist2_body(x_ref, w_ref, y_ref, y8_ref, out_ref):
    a = x_ref[0]
    yb = y_ref[0]
    aa = jnp.sum(a * a, axis=1, keepdims=True)
    ab = jax.lax.dot_general(
        w_ref[0], y8_ref[0],
        (((1,), (0,)), ((), ())),
        preferred_element_type=jnp.float32,
    )
    t = ab + aa
    tmin = jnp.min(t, axis=0, keepdims=True)
    bb = jnp.sum(yb * yb, axis=0, keepdims=True)
    out_ref[...] = jnp.maximum(tmin + bb, 0.0)[None]


def _loss_body(d2_ref, y0_ref, y1_ref, sdyn_ref, cdyn_ref, sst_ref, cst_ref):
    d2 = d2_ref[...]
    pos = d2 < THRES_DIST
    neg = d2 > THRES_STATIC
    sq = jnp.zeros_like(d2)
    sst = jnp.float32(0.0)
    cst = jnp.float32(0.0)
    for c in range(3):
        diff = y1_ref[c * B:(c + 1) * B, :] - y0_ref[c * B:(c + 1) * B, :]
        sq = sq + diff * diff
        sel_st = jnp.logical_and(neg, diff > 0.0)
        sst = sst + jnp.sum(jnp.where(sel_st, diff * diff, 0.0))
        cst = cst + jnp.sum(sel_st.astype(jnp.float32))
    sel_dyn = jnp.logical_and(pos, sq > 0.0)
    norm = jnp.sqrt(jnp.where(sel_dyn, sq, 1.0))
    z = 0.1 - norm
    sp = jnp.maximum(z, 0.0) + jnp.log1p(jnp.exp(-jnp.abs(z)))
    sdyn_ref[0, 0] = jnp.sum(jnp.where(sel_dyn, sp, 0.0))
    cdyn_ref[0, 0] = jnp.sum(sel_dyn.astype(jnp.float32))
    sst_ref[0, 0] = sst
    cst_ref[0, 0] = cst


@jax.jit
def kernel(x, y_hat0, y_hat1):
    xp = x[:, 0, :, :3].astype(jnp.float32)
    y0p = y_hat0[:, 0].astype(jnp.float32)
    yt = jnp.transpose(y0p, (0, 2, 1))
    wp = jnp.zeros((B, N, 8), jnp.bfloat16).at[:, :, :3].set(
        (-2.0 * xp).astype(jnp.bfloat16))
    y8 = jnp.zeros((B, 8, M), jnp.bfloat16).at[:, :3, :].set(
        yt.astype(jnp.bfloat16))

    d2 = pl.pallas_call(
        _dist2_body,
        grid=(B, M // MT),
        in_specs=[
            pl.BlockSpec((1, N, 3), lambda b, m: (b, 0, 0)),
            pl.BlockSpec((1, N, 8), lambda b, m: (b, 0, 0)),
            pl.BlockSpec((1, 3, MT), lambda b, m: (b, 0, m)),
            pl.BlockSpec((1, 8, MT), lambda b, m: (b, 0, m)),
        ],
        out_specs=pl.BlockSpec((1, 1, MT), lambda b, m: (b, 0, m)),
        out_shape=jax.ShapeDtypeStruct((B, 1, M), jnp.float32),
    )(xp, wp, yt, y8)
    d2 = d2[:, 0, :]

    y0r = jnp.transpose(y0p, (2, 0, 1)).reshape(3 * B, M)
    y1r = jnp.transpose(y_hat1[:, 0].astype(jnp.float32), (2, 0, 1)).reshape(3 * B, M)

    scal = jax.ShapeDtypeStruct((1, 1), jnp.float32)
    sspec = pl.BlockSpec(memory_space=pltpu.SMEM)
    sdyn, cdyn, sst, cst = pl.pallas_call(
        _loss_body,
        out_shape=(scal, scal, scal, scal),
        out_specs=(sspec, sspec, sspec, sspec),
    )(d2, y0r, y1r)

    loss_dynamic = (sdyn[0, 0] / cdyn[0, 0]).astype(jnp.float32)
    loss_static = (sst[0, 0] / cst[0, 0]).astype(jnp.float32)
    return (loss_dynamic, loss_static)


# speedup vs baseline: 2.5809x; 1.0259x over previous
import functools

import jax
import jax.numpy as jnp
from jax.experimental import pallas as pl
from jax.experimental.pallas import tpu as pltpu

THRES_STATIC = 0.001
THRES_DIST = 0.002

B, N, M = 4, 4096, 4096
MT = 2048


def _dist2_body(x_ref, w_ref, y_ref, y8_ref, out_ref):
    a = x_ref[0]
    yb = y_ref[0]
    aa = jnp.sum(a * a, axis=1, keepdims=True)
    ab = jax.lax.dot_general(
        w_ref[0], y8_ref[0],
        (((1,), (0,)), ((), ())),
        preferred_element_type=jnp.float32,
    )
    t = ab + aa
    tmin = jnp.min(t, axis=0, keepdims=True)
    bb = jnp.sum(yb * yb, axis=0, keepdims=True)
    out_ref[...] = jnp.maximum(tmin + bb, 0.0)[None]


def _loss_body(d2_ref, y0_ref, y1_ref, sdyn_ref, cdyn_ref, sst_ref, cst_ref):
    d2 = d2_ref[...]
    pos = d2 < THRES_DIST
    neg = d2 > THRES_STATIC
    sq = jnp.zeros_like(d2)
    sst = jnp.float32(0.0)
    cst = jnp.float32(0.0)
    for c in range(3):
        diff = y1_ref[c * B:(c + 1) * B, :] - y0_ref[c * B:(c + 1) * B, :]
        sq = sq + diff * diff
        sel_st = jnp.logical_and(neg, diff > 0.0)
        sst = sst + jnp.sum(jnp.where(sel_st, diff * diff, 0.0))
        cst = cst + jnp.sum(sel_st.astype(jnp.float32))
    sel_dyn = jnp.logical_and(pos, sq > 0.0)
    norm = jnp.sqrt(jnp.where(sel_dyn, sq, 1.0))
    z = 0.1 - norm
    sp = jnp.maximum(z, 0.0) + jnp.log1p(jnp.exp(-jnp.abs(z)))
    sdyn_ref[0, 0] = jnp.sum(jnp.where(sel_dyn, sp, 0.0))
    cdyn_ref[0, 0] = jnp.sum(sel_dyn.astype(jnp.float32))
    sst_ref[0, 0] = sst
    cst_ref[0, 0] = cst


@jax.jit
def kernel(x, y_hat0, y_hat1):
    xp = x[:, 0, :, :3].astype(jnp.float32)
    y0p = y_hat0[:, 0].astype(jnp.float32)
    yt = jnp.transpose(y0p, (0, 2, 1))
    wp = jnp.zeros((B, N, 8), jnp.bfloat16).at[:, :, :3].set(
        (-2.0 * xp).astype(jnp.bfloat16))
    y8 = jnp.zeros((B, 8, M), jnp.bfloat16).at[:, :3, :].set(
        yt.astype(jnp.bfloat16))

    d2 = pl.pallas_call(
        _dist2_body,
        grid=(B, M // MT),
        in_specs=[
            pl.BlockSpec((1, N, 3), lambda b, m: (b, 0, 0)),
            pl.BlockSpec((1, N, 8), lambda b, m: (b, 0, 0)),
            pl.BlockSpec((1, 3, MT), lambda b, m: (b, 0, m)),
            pl.BlockSpec((1, 8, MT), lambda b, m: (b, 0, m)),
        ],
        out_specs=pl.BlockSpec((1, 1, MT), lambda b, m: (b, 0, m)),
        out_shape=jax.ShapeDtypeStruct((B, 1, M), jnp.float32),
    )(xp, wp, yt, y8)
    d2 = d2[:, 0, :]

    y0r = jnp.transpose(y0p, (2, 0, 1)).reshape(3 * B, M)
    y1r = jnp.transpose(y_hat1[:, 0].astype(jnp.float32), (2, 0, 1)).reshape(3 * B, M)

    scal = jax.ShapeDtypeStruct((1, 1), jnp.float32)
    sspec = pl.BlockSpec(memory_space=pltpu.SMEM)
    sdyn, cdyn, sst, cst = pl.pallas_call(
        _loss_body,
        out_shape=(scal, scal, scal, scal),
        out_specs=(sspec, sspec, sspec, sspec),
    )(d2, y0r, y1r)

    loss_dynamic = (sdyn[0, 0] / cdyn[0, 0]).astype(jnp.float32)
    loss_static = (sst[0, 0] / cst[0, 0]).astype(jnp.float32)
    return (loss_dynamic, loss_static)


# speedup vs baseline: 2.6043x; 1.0091x over previous
import functools

import jax
import jax.numpy as jnp
from jax.experimental import pallas as pl
from jax.experimental.pallas import tpu as pltpu

THRES_STATIC = 0.001
THRES_DIST = 0.002

B, N, M = 4, 4096, 4096
MT = 4096


def _dist2_body(x_ref, w_ref, y_ref, y8_ref, out_ref):
    a = x_ref[0]
    yb = y_ref[0]
    aa = jnp.sum(a * a, axis=1, keepdims=True)
    ab = jax.lax.dot_general(
        w_ref[0], y8_ref[0],
        (((1,), (0,)), ((), ())),
        preferred_element_type=jnp.float32,
    )
    t = ab + aa
    tmin = jnp.min(t, axis=0, keepdims=True)
    bb = jnp.sum(yb * yb, axis=0, keepdims=True)
    out_ref[...] = jnp.maximum(tmin + bb, 0.0)[None]


def _loss_body(d2_ref, y0_ref, y1_ref, sdyn_ref, cdyn_ref, sst_ref, cst_ref):
    d2 = d2_ref[...]
    pos = d2 < THRES_DIST
    neg = d2 > THRES_STATIC
    sq = jnp.zeros_like(d2)
    sst = jnp.float32(0.0)
    cst = jnp.float32(0.0)
    for c in range(3):
        diff = y1_ref[c * B:(c + 1) * B, :] - y0_ref[c * B:(c + 1) * B, :]
        sq = sq + diff * diff
        sel_st = jnp.logical_and(neg, diff > 0.0)
        sst = sst + jnp.sum(jnp.where(sel_st, diff * diff, 0.0))
        cst = cst + jnp.sum(sel_st.astype(jnp.float32))
    sel_dyn = jnp.logical_and(pos, sq > 0.0)
    norm = jnp.sqrt(jnp.where(sel_dyn, sq, 1.0))
    z = 0.1 - norm
    sp = jnp.maximum(z, 0.0) + jnp.log1p(jnp.exp(-jnp.abs(z)))
    sdyn_ref[0, 0] = jnp.sum(jnp.where(sel_dyn, sp, 0.0))
    cdyn_ref[0, 0] = jnp.sum(sel_dyn.astype(jnp.float32))
    sst_ref[0, 0] = sst
    cst_ref[0, 0] = cst


@jax.jit
def kernel(x, y_hat0, y_hat1):
    xp = x[:, 0, :, :3].astype(jnp.float32)
    y0p = y_hat0[:, 0].astype(jnp.float32)
    yt = jnp.transpose(y0p, (0, 2, 1))
    wp = jnp.zeros((B, N, 8), jnp.bfloat16).at[:, :, :3].set(
        (-2.0 * xp).astype(jnp.bfloat16))
    y8 = jnp.zeros((B, 8, M), jnp.bfloat16).at[:, :3, :].set(
        yt.astype(jnp.bfloat16))

    d2 = pl.pallas_call(
        _dist2_body,
        grid=(B, M // MT),
        in_specs=[
            pl.BlockSpec((1, N, 3), lambda b, m: (b, 0, 0)),
            pl.BlockSpec((1, N, 8), lambda b, m: (b, 0, 0)),
            pl.BlockSpec((1, 3, MT), lambda b, m: (b, 0, m)),
            pl.BlockSpec((1, 8, MT), lambda b, m: (b, 0, m)),
        ],
        out_specs=pl.BlockSpec((1, 1, MT), lambda b, m: (b, 0, m)),
        out_shape=jax.ShapeDtypeStruct((B, 1, M), jnp.float32),
    )(xp, wp, yt, y8)
    d2 = d2[:, 0, :]

    y0r = jnp.transpose(y0p, (2, 0, 1)).reshape(3 * B, M)
    y1r = jnp.transpose(y_hat1[:, 0].astype(jnp.float32), (2, 0, 1)).reshape(3 * B, M)

    scal = jax.ShapeDtypeStruct((1, 1), jnp.float32)
    sspec = pl.BlockSpec(memory_space=pltpu.SMEM)
    sdyn, cdyn, sst, cst = pl.pallas_call(
        _loss_body,
        out_shape=(scal, scal, scal, scal),
        out_specs=(sspec, sspec, sspec, sspec),
    )(d2, y0r, y1r)

    loss_dynamic = (sdyn[0, 0] / cdyn[0, 0]).astype(jnp.float32)
    loss_static = (sst[0, 0] / cst[0, 0]).astype(jnp.float32)
    return (loss_dynamic, loss_static)
